# Initial kernel scaffold; baseline (speedup 1.0000x reference)
import jax
import jax.numpy as jnp
from jax.experimental import pallas as pl


def kernel(hidden_states, weight):
    raise NotImplementedError("write your pallas kernel here")


# speedup vs baseline: 2.8234x; 2.8234x over previous
import functools

import jax
import jax.numpy as jnp
from jax.experimental import pallas as pl

TOP_K = 8
ALPHA = 0.1


def _gate_kernel(hs_ref, w_ref, idx_ref, wgt_ref, counts_ref, ssum_ref,
                 loads_ref, aux_ref, vio_ref, imb_ref, util_ref, ratio_ref,
                 *, blocks_per_batch, num_blocks, seq, e):
    i = pl.program_id(0)
    b = i // blocks_per_batch

    @pl.when(i == 0)
    def _init():
        counts_ref[...] = jnp.zeros_like(counts_ref)
        ssum_ref[...] = jnp.zeros_like(ssum_ref)

    hs = hs_ref[...]
    w = w_ref[...]
    logits = jax.lax.dot_general(
        hs, w, (((1,), (1,)), ((), ())), preferred_element_type=jnp.float32)
    m = jnp.max(logits, axis=1, keepdims=True)
    ex = jnp.exp(logits - m)
    scores = ex / jnp.sum(ex, axis=1, keepdims=True)

    iota = jax.lax.broadcasted_iota(jnp.int32, scores.shape, 1)
    bits = jax.lax.bitcast_convert_type(scores, jnp.int32)
    keys = (bits & jnp.int32(~(e - 1))) | (e - 1 - iota)
    minkey = jnp.int32(-2**31)
    vals = []
    idxs = []
    for _ in range(TOP_K):
        kmax = jnp.max(keys, axis=1, keepdims=True)
        keys = jnp.where(keys == kmax, minkey, keys)
        idxs.append(e - 1 - (kmax & (e - 1)))
        vals.append(jax.lax.bitcast_convert_type(
            kmax & jnp.int32(~(e - 1)), jnp.float32))

    topw = jnp.concatenate(vals, axis=1)
    topi = jnp.concatenate(idxs, axis=1)
    denom = jnp.sum(topw, axis=1, keepdims=True) + 1e-20
    wgt_ref[...] = topw / denom
    idx_ref[...] = topi

    selected = (keys < 0).astype(jnp.float32)
    cnt_part = jnp.sum(selected, axis=0, keepdims=True)
    ssum_part = jnp.sum(scores, axis=0, keepdims=True)
    counts_ref[pl.ds(b, 1), :] += cnt_part
    ssum_ref[pl.ds(b, 1), :] += ssum_part

    @pl.when(i == num_blocks - 1)
    def _finish():
        counts = counts_ref[...]
        ssum = ssum_ref[...]
        loads = jnp.sum(counts, axis=0, keepdims=True)
        loads_ref[...] = loads
        total = jnp.sum(loads)
        expected = total / e
        mean = total / e
        maxl = jnp.max(loads)
        vio_ref[...] = ((maxl - expected) / expected).reshape(1, 1)
        var = jnp.sum((loads - mean) ** 2) / (e - 1)
        imb_ref[...] = (jnp.sqrt(var) / mean).reshape(1, 1)
        util_ref[...] = (jnp.sum((loads > 0).astype(jnp.float32)) / e).reshape(1, 1)
        minl = jnp.min(jnp.where(loads > 0, loads, jnp.inf))
        ratio_ref[...] = (maxl / minl).reshape(1, 1)
        bsz = counts.shape[0]
        ce = counts / (seq * TOP_K / e)
        smean = ssum / seq
        aux_ref[...] = (jnp.sum(ce * smean) / bsz * ALPHA).reshape(1, 1)


def kernel(hidden_states, weight):
    bsz, seq, h = hidden_states.shape
    e = weight.shape[0]
    blk = 512 if seq % 512 == 0 else seq
    blocks_per_batch = seq // blk
    num_blocks = bsz * blocks_per_batch
    hs = hidden_states.reshape(bsz * seq, h)

    out_shapes = (
        jax.ShapeDtypeStruct((bsz * seq, TOP_K), jnp.int32),
        jax.ShapeDtypeStruct((bsz * seq, TOP_K), jnp.float32),
        jax.ShapeDtypeStruct((bsz, e), jnp.float32),
        jax.ShapeDtypeStruct((bsz, e), jnp.float32),
        jax.ShapeDtypeStruct((1, e), jnp.float32),
        jax.ShapeDtypeStruct((1, 1), jnp.float32),
        jax.ShapeDtypeStruct((1, 1), jnp.float32),
        jax.ShapeDtypeStruct((1, 1), jnp.float32),
        jax.ShapeDtypeStruct((1, 1), jnp.float32),
        jax.ShapeDtypeStruct((1, 1), jnp.float32),
    )
    res = pl.pallas_call(
        functools.partial(_gate_kernel, blocks_per_batch=blocks_per_batch,
                          num_blocks=num_blocks, seq=seq, e=e),
        grid=(num_blocks,),
        in_specs=[
            pl.BlockSpec((blk, h), lambda i: (i, 0)),
            pl.BlockSpec((e, h), lambda i: (0, 0)),
        ],
        out_specs=(
            pl.BlockSpec((blk, TOP_K), lambda i: (i, 0)),
            pl.BlockSpec((blk, TOP_K), lambda i: (i, 0)),
            pl.BlockSpec((bsz, e), lambda i: (0, 0)),
            pl.BlockSpec((bsz, e), lambda i: (0, 0)),
            pl.BlockSpec((1, e), lambda i: (0, 0)),
            pl.BlockSpec((1, 1), lambda i: (0, 0)),
            pl.BlockSpec((1, 1), lambda i: (0, 0)),
            pl.BlockSpec((1, 1), lambda i: (0, 0)),
            pl.BlockSpec((1, 1), lambda i: (0, 0)),
            pl.BlockSpec((1, 1), lambda i: (0, 0)),
        ),
        out_shape=out_shapes,
    )(hs, weight)
    (topi, topw, _counts, _ssum, loads, aux, vio, imb, util, ratio) = res
    return (
        topi,
        topw,
        aux[0, 0],
        loads[0],
        vio[0, 0],
        imb[0, 0],
        util[0, 0],
        ratio[0, 0],
    )


# speedup vs baseline: 3.1119x; 1.1022x over previous
import functools

import jax
import jax.numpy as jnp
from jax.experimental import pallas as pl

TOP_K = 8
ALPHA = 0.1


def _gate_kernel(hs_ref, w_ref, idx_ref, wgt_ref, counts_ref, ssum_ref,
                 loads_ref, aux_ref, vio_ref, imb_ref, util_ref, ratio_ref,
                 *, blocks_per_batch, num_blocks, seq, e):
    i = pl.program_id(0)
    b = i // blocks_per_batch

    @pl.when(i == 0)
    def _init():
        counts_ref[...] = jnp.zeros_like(counts_ref)
        ssum_ref[...] = jnp.zeros_like(ssum_ref)

    hs = hs_ref[...]
    w = w_ref[...]
    logits = jax.lax.dot_general(
        hs, w, (((1,), (1,)), ((), ())), preferred_element_type=jnp.float32)
    m = jnp.max(logits, axis=1, keepdims=True)
    ex = jnp.exp(logits - m)
    scores = ex / jnp.sum(ex, axis=1, keepdims=True)

    iota = jax.lax.broadcasted_iota(jnp.int32, scores.shape, 1)
    bits = jax.lax.bitcast_convert_type(scores, jnp.int32)
    keys = (bits & jnp.int32(~(e - 1))) | (e - 1 - iota)
    minkey = jnp.int32(-2**31)
    vals = []
    idxs = []
    for _ in range(TOP_K):
        kmax = jnp.max(keys, axis=1, keepdims=True)
        keys = jnp.where(keys == kmax, minkey, keys)
        idxs.append(e - 1 - (kmax & (e - 1)))
        vals.append(jax.lax.bitcast_convert_type(
            kmax & jnp.int32(~(e - 1)), jnp.float32))

    topw = jnp.concatenate(vals, axis=1)
    topi = jnp.concatenate(idxs, axis=1)
    denom = jnp.sum(topw, axis=1, keepdims=True) + 1e-20
    wgt_ref[...] = topw / denom
    idx_ref[...] = topi

    selected = (keys < 0).astype(jnp.float32)
    cnt_part = jnp.sum(selected, axis=0, keepdims=True)
    ssum_part = jnp.sum(scores, axis=0, keepdims=True)
    counts_ref[pl.ds(b, 1), :] += cnt_part
    ssum_ref[pl.ds(b, 1), :] += ssum_part

    @pl.when(i == num_blocks - 1)
    def _finish():
        counts = counts_ref[...]
        ssum = ssum_ref[...]
        loads = jnp.sum(counts, axis=0, keepdims=True)
        loads_ref[...] = loads
        total = jnp.sum(loads)
        expected = total / e
        mean = total / e
        maxl = jnp.max(loads)
        vio_ref[...] = ((maxl - expected) / expected).reshape(1, 1)
        var = jnp.sum((loads - mean) ** 2) / (e - 1)
        imb_ref[...] = (jnp.sqrt(var) / mean).reshape(1, 1)
        util_ref[...] = (jnp.sum((loads > 0).astype(jnp.float32)) / e).reshape(1, 1)
        minl = jnp.min(jnp.where(loads > 0, loads, jnp.inf))
        ratio_ref[...] = (maxl / minl).reshape(1, 1)
        bsz = counts.shape[0]
        ce = counts / (seq * TOP_K / e)
        smean = ssum / seq
        aux_ref[...] = (jnp.sum(ce * smean) / bsz * ALPHA).reshape(1, 1)


def kernel(hidden_states, weight):
    bsz, seq, h = hidden_states.shape
    e = weight.shape[0]
    blk = 1024 if seq % 1024 == 0 else seq
    blocks_per_batch = seq // blk
    num_blocks = bsz * blocks_per_batch
    hs = hidden_states.reshape(bsz * seq, h)

    out_shapes = (
        jax.ShapeDtypeStruct((bsz * seq, TOP_K), jnp.int32),
        jax.ShapeDtypeStruct((bsz * seq, TOP_K), jnp.float32),
        jax.ShapeDtypeStruct((bsz, e), jnp.float32),
        jax.ShapeDtypeStruct((bsz, e), jnp.float32),
        jax.ShapeDtypeStruct((1, e), jnp.float32),
        jax.ShapeDtypeStruct((1, 1), jnp.float32),
        jax.ShapeDtypeStruct((1, 1), jnp.float32),
        jax.ShapeDtypeStruct((1, 1), jnp.float32),
        jax.ShapeDtypeStruct((1, 1), jnp.float32),
        jax.ShapeDtypeStruct((1, 1), jnp.float32),
    )
    res = pl.pallas_call(
        functools.partial(_gate_kernel, blocks_per_batch=blocks_per_batch,
                          num_blocks=num_blocks, seq=seq, e=e),
        grid=(num_blocks,),
        in_specs=[
            pl.BlockSpec((blk, h), lambda i: (i, 0)),
            pl.BlockSpec((e, h), lambda i: (0, 0)),
        ],
        out_specs=(
            pl.BlockSpec((blk, TOP_K), lambda i: (i, 0)),
            pl.BlockSpec((blk, TOP_K), lambda i: (i, 0)),
            pl.BlockSpec((bsz, e), lambda i: (0, 0)),
            pl.BlockSpec((bsz, e), lambda i: (0, 0)),
            pl.BlockSpec((1, e), lambda i: (0, 0)),
            pl.BlockSpec((1, 1), lambda i: (0, 0)),
            pl.BlockSpec((1, 1), lambda i: (0, 0)),
            pl.BlockSpec((1, 1), lambda i: (0, 0)),
            pl.BlockSpec((1, 1), lambda i: (0, 0)),
            pl.BlockSpec((1, 1), lambda i: (0, 0)),
        ),
        out_shape=out_shapes,
    )(hs, weight)
    (topi, topw, _counts, _ssum, loads, aux, vio, imb, util, ratio) = res
    return (
        topi,
        topw,
        aux[0, 0],
        loads[0],
        vio[0, 0],
        imb[0, 0],
        util[0, 0],
        ratio[0, 0],
    )
